# baseline (speedup 1.0000x reference)
import jax
import jax.numpy as jnp
from jax.experimental import pallas as pl
from jax.experimental.pallas import tpu as pltpu

_N_TOK = 16384
_D_IN = 2048
_D_HID = 256
_N_EXP = 16
_BT = 1024


def _router_body(xa_ref, xb_ref, xc_ref, xd_ref, w1a_ref, w1b_ref, w1c_ref, w1d_ref, b1_ref, w2_ref, b2_ref,
                 w_out_ref, l_out_ref):
    h = (jnp.dot(xa_ref[...], w1a_ref[...], preferred_element_type=jnp.float32)
         + jnp.dot(xb_ref[...], w1b_ref[...], preferred_element_type=jnp.float32)
         + jnp.dot(xc_ref[...], w1c_ref[...], preferred_element_type=jnp.float32)
         + jnp.dot(xd_ref[...], w1d_ref[...], preferred_element_type=jnp.float32)
         + b1_ref[...])
    h = 0.5 * h * (1.0 + jax.lax.erf(h * 0.7071067811865476))
    logits = jnp.dot(h, w2_ref[...], preferred_element_type=jnp.float32) + b2_ref[...]
    l_out_ref[...] = logits

    logitsT = logits.T
    laneT = jax.lax.broadcasted_iota(jnp.int32, logitsT.shape, 0).astype(jnp.float32)
    m1 = jnp.max(logitsT, axis=0, keepdims=True)
    idx1 = jnp.min(jnp.where(logitsT == m1, laneT, _N_EXP), axis=0, keepdims=True)
    is1 = laneT == idx1
    masked = jnp.where(is1, -jnp.inf, logitsT)
    m2 = jnp.max(masked, axis=0, keepdims=True)
    idx2 = jnp.min(jnp.where(masked == m2, laneT, _N_EXP), axis=0, keepdims=True)
    is2 = laneT == idx2
    g1 = jax.nn.sigmoid(m1 - m2)
    wT = jnp.where(is1, g1, 0.0) + jnp.where(is2, 1.0 - g1, 0.0)
    w_out_ref[...] = wT.T


def kernel(router_input, W1, b1, W2, b2):
    b1 = b1.reshape(1, _D_HID)
    b2 = b2.reshape(1, _N_EXP)
    grid = (_N_TOK // _BT,)
    weights, logits = pl.pallas_call(
        _router_body,
        grid=grid,
        in_specs=[
            pl.BlockSpec((_BT, _D_IN // 4), lambda i: (i, 0)),
            pl.BlockSpec((_BT, _D_IN // 4), lambda i: (i, 1)),
            pl.BlockSpec((_BT, _D_IN // 4), lambda i: (i, 2)),
            pl.BlockSpec((_BT, _D_IN // 4), lambda i: (i, 3)),
            pl.BlockSpec((_D_IN // 4, _D_HID), lambda i: (0, 0)),
            pl.BlockSpec((_D_IN // 4, _D_HID), lambda i: (1, 0)),
            pl.BlockSpec((_D_IN // 4, _D_HID), lambda i: (2, 0)),
            pl.BlockSpec((_D_IN // 4, _D_HID), lambda i: (3, 0)),
            pl.BlockSpec((1, _D_HID), lambda i: (0, 0)),
            pl.BlockSpec((_D_HID, _N_EXP), lambda i: (0, 0)),
            pl.BlockSpec((1, _N_EXP), lambda i: (0, 0)),
        ],
        out_specs=[
            pl.BlockSpec((_BT, _N_EXP), lambda i: (i, 0)),
            pl.BlockSpec((_BT, _N_EXP), lambda i: (i, 0)),
        ],
        out_shape=[
            jax.ShapeDtypeStruct((_N_TOK, _N_EXP), jnp.float32),
            jax.ShapeDtypeStruct((_N_TOK, _N_EXP), jnp.float32),
        ],
        compiler_params=pltpu.CompilerParams(
            dimension_semantics=("parallel",),
        ),
    )(router_input, router_input, router_input, router_input, W1, W1, W1, W1, b1, W2, b2)
    return (weights, logits)


# speedup vs baseline: 1.0370x; 1.0370x over previous
import jax
import jax.numpy as jnp
from jax.experimental import pallas as pl
from jax.experimental.pallas import tpu as pltpu

_N_TOK = 16384
_D_IN = 2048
_D_HID = 256
_N_EXP = 16
_BT = 2048


def _router_body(x_ref, w1_ref, b1_ref, w2_ref, b2_ref,
                 w_out_ref, l_out_ref):
    h = jnp.dot(x_ref[...], w1_ref[...], preferred_element_type=jnp.float32) + b1_ref[...]
    h = 0.5 * h * (1.0 + jax.lax.erf(h * 0.7071067811865476))
    logits = jnp.dot(h, w2_ref[...], preferred_element_type=jnp.float32) + b2_ref[...]
    l_out_ref[...] = logits

    logitsT = logits.T
    laneT = jax.lax.broadcasted_iota(jnp.int32, logitsT.shape, 0).astype(jnp.float32)
    m1 = jnp.max(logitsT, axis=0, keepdims=True)
    idx1 = jnp.min(jnp.where(logitsT == m1, laneT, _N_EXP), axis=0, keepdims=True)
    is1 = laneT == idx1
    masked = jnp.where(is1, -jnp.inf, logitsT)
    m2 = jnp.max(masked, axis=0, keepdims=True)
    idx2 = jnp.min(jnp.where(masked == m2, laneT, _N_EXP), axis=0, keepdims=True)
    is2 = laneT == idx2
    g1 = jax.nn.sigmoid(m1 - m2)
    wT = jnp.where(is1, g1, 0.0) + jnp.where(is2, 1.0 - g1, 0.0)
    w_out_ref[...] = wT.T


def kernel(router_input, W1, b1, W2, b2):
    b1 = b1.reshape(1, _D_HID)
    b2 = b2.reshape(1, _N_EXP)
    grid = (_N_TOK // _BT,)
    weights, logits = pl.pallas_call(
        _router_body,
        grid=grid,
        in_specs=[
            pl.BlockSpec((_BT, _D_IN), lambda i: (i, 0)),
            pl.BlockSpec((_D_IN, _D_HID), lambda i: (0, 0)),
            pl.BlockSpec((1, _D_HID), lambda i: (0, 0)),
            pl.BlockSpec((_D_HID, _N_EXP), lambda i: (0, 0)),
            pl.BlockSpec((1, _N_EXP), lambda i: (0, 0)),
        ],
        out_specs=[
            pl.BlockSpec((_BT, _N_EXP), lambda i: (i, 0)),
            pl.BlockSpec((_BT, _N_EXP), lambda i: (i, 0)),
        ],
        out_shape=[
            jax.ShapeDtypeStruct((_N_TOK, _N_EXP), jnp.float32),
            jax.ShapeDtypeStruct((_N_TOK, _N_EXP), jnp.float32),
        ],
        compiler_params=pltpu.CompilerParams(
            dimension_semantics=("parallel",),
        ),
    )(router_input, W1, b1, W2, b2)
    return (weights, logits)


# speedup vs baseline: 1.0400x; 1.0028x over previous
import jax
import jax.numpy as jnp
from jax.experimental import pallas as pl
from jax.experimental.pallas import tpu as pltpu

_N_TOK = 16384
_D_IN = 2048
_D_HID = 256
_N_EXP = 16
_BT = 2048


def _router_body(xa_ref, xb_ref, w1a_ref, w1b_ref, b1_ref, w2_ref, b2_ref,
                 w_out_ref, l_out_ref):
    h = (jnp.dot(xa_ref[...], w1a_ref[...], preferred_element_type=jnp.float32)
         + jnp.dot(xb_ref[...], w1b_ref[...], preferred_element_type=jnp.float32)
         + b1_ref[...])
    h = 0.5 * h * (1.0 + jax.lax.erf(h * 0.7071067811865476))
    logits = jnp.dot(h, w2_ref[...], preferred_element_type=jnp.float32) + b2_ref[...]
    l_out_ref[...] = logits

    logitsT = logits.T
    laneT = jax.lax.broadcasted_iota(jnp.int32, logitsT.shape, 0).astype(jnp.float32)
    m1 = jnp.max(logitsT, axis=0, keepdims=True)
    idx1 = jnp.min(jnp.where(logitsT == m1, laneT, _N_EXP), axis=0, keepdims=True)
    is1 = laneT == idx1
    masked = jnp.where(is1, -jnp.inf, logitsT)
    m2 = jnp.max(masked, axis=0, keepdims=True)
    idx2 = jnp.min(jnp.where(masked == m2, laneT, _N_EXP), axis=0, keepdims=True)
    is2 = laneT == idx2
    g1 = jax.nn.sigmoid(m1 - m2)
    wT = jnp.where(is1, g1, 0.0) + jnp.where(is2, 1.0 - g1, 0.0)
    w_out_ref[...] = wT.T


def kernel(router_input, W1, b1, W2, b2):
    b1 = b1.reshape(1, _D_HID)
    b2 = b2.reshape(1, _N_EXP)
    grid = (_N_TOK // _BT,)
    weights, logits = pl.pallas_call(
        _router_body,
        grid=grid,
        in_specs=[
            pl.BlockSpec((_BT, _D_IN // 2), lambda i: (i, 0)),
            pl.BlockSpec((_BT, _D_IN // 2), lambda i: (i, 1)),
            pl.BlockSpec((_D_IN // 2, _D_HID), lambda i: (0, 0)),
            pl.BlockSpec((_D_IN // 2, _D_HID), lambda i: (1, 0)),
            pl.BlockSpec((1, _D_HID), lambda i: (0, 0)),
            pl.BlockSpec((_D_HID, _N_EXP), lambda i: (0, 0)),
            pl.BlockSpec((1, _N_EXP), lambda i: (0, 0)),
        ],
        out_specs=[
            pl.BlockSpec((_BT, _N_EXP), lambda i: (i, 0)),
            pl.BlockSpec((_BT, _N_EXP), lambda i: (i, 0)),
        ],
        out_shape=[
            jax.ShapeDtypeStruct((_N_TOK, _N_EXP), jnp.float32),
            jax.ShapeDtypeStruct((_N_TOK, _N_EXP), jnp.float32),
        ],
        compiler_params=pltpu.CompilerParams(
            dimension_semantics=("parallel",),
        ),
    )(router_input, router_input, W1, W1, b1, W2, b2)
    return (weights, logits)
